# baseline (speedup 1.0000x reference)
import functools

import jax
import jax.numpy as jnp
from jax import lax
from jax.experimental import pallas as pl
from jax.experimental.pallas import tpu as pltpu
from jax.experimental.pallas import tpu_sc as plsc

_NC = 2
_NS = 16
_SUB = 16
_LANES = 16


def _make_sc_kernel(b, l, d, dtype):
    nw = _NC * _NS
    l_per_w = l // nw
    nsub = l_per_w // _SUB
    words = _SUB * d

    mesh = plsc.VectorSubcoreMesh(core_axis_name="c", subcore_axis_name="s")

    @functools.partial(
        pl.kernel,
        mesh=mesh,
        out_type=jax.ShapeDtypeStruct((b * l * d,), dtype),
        scratch_types=[
            pltpu.VMEM((words,), dtype),
            pltpu.VMEM((words,), dtype),
        ],
    )
    def k(x_hbm, pe_hbm, out_hbm, pe_buf, x_buf):
        c = lax.axis_index("c")
        s = lax.axis_index("s")
        wid = s * _NC + c
        lbase = wid * l_per_w

        def sub_body(j, _):
            l0 = lbase + j * _SUB
            pltpu.sync_copy(pe_hbm.at[pl.ds(l0 * d, words)], pe_buf)

            def batch_body(bi, _):
                r0 = (bi * l + l0) * d
                pltpu.sync_copy(x_hbm.at[pl.ds(r0, words)], x_buf)

                @plsc.parallel_loop(0, words, _LANES, unroll=8)
                def grp(off):
                    v = pe_buf[pl.ds(off, _LANES)]
                    plsc.addupdate(x_buf.at[pl.ds(off, _LANES)], v)

                pltpu.sync_copy(x_buf, out_hbm.at[pl.ds(r0, words)])
                return 0

            lax.fori_loop(0, b, batch_body, 0, unroll=False)
            return 0

        lax.fori_loop(0, nsub, sub_body, 0, unroll=False)

    return k


def kernel(x, pe_weight):
    b, l, d = x.shape
    xf = x.reshape(b * l * d)
    pef = pe_weight.reshape(pe_weight.shape[0] * d)
    out = _make_sc_kernel(b, l, d, x.dtype)(xf, pef)
    return out.reshape(b, l, d)


# speedup vs baseline: 6.0716x; 6.0716x over previous
import functools

import jax
import jax.numpy as jnp
from jax.experimental import pallas as pl

_TL = 2048


def _pe_add_kernel(x_ref, pe_ref, o_ref, *, blocks_per_l: int):
    i = pl.program_id(0)
    off = (i % blocks_per_l) * _TL
    o_ref[...] = x_ref[...] + pe_ref[pl.ds(off, _TL), :]


def kernel(x, pe_weight):
    b, l, d = x.shape
    xf = x.reshape(b * l, d)
    out = pl.pallas_call(
        functools.partial(_pe_add_kernel, blocks_per_l=l // _TL),
        grid=(b * l // _TL,),
        in_specs=[
            pl.BlockSpec((_TL, d), lambda i: (i, 0)),
            pl.BlockSpec((l, d), lambda i: (0, 0)),
        ],
        out_specs=pl.BlockSpec((_TL, d), lambda i: (i, 0)),
        out_shape=jax.ShapeDtypeStruct((b * l, d), x.dtype),
    )(xf, pe_weight)
    return out.reshape(b, l, d)
